# baseline (speedup 1.0000x reference)
import functools

import jax
import jax.numpy as jnp
from jax import lax
from jax.experimental import pallas as pl
from jax.experimental.pallas import tpu as pltpu
from jax.experimental.pallas import tpu_sc as plsc


def _sc_gather(table, idx):
    V, D = table.shape
    (B,) = idx.shape
    info = plsc.get_sparse_core_info()
    nw = info.num_cores * info.num_subcores
    b_per_w = B // nw
    mesh = plsc.VectorSubcoreMesh(core_axis_name="c", subcore_axis_name="s")

    @functools.partial(
        pl.kernel,
        mesh=mesh,
        out_type=jax.ShapeDtypeStruct((B, D), jnp.float32),
        scratch_types=[
            pltpu.VMEM((b_per_w,), jnp.int32),
            pltpu.VMEM((b_per_w, D), jnp.float32),
            pltpu.SemaphoreType.DMA,
        ],
    )
    def gather_kernel(table_hbm, idx_hbm, out_hbm, idx_v, rows_v, sem):
        wid = lax.axis_index("s") * info.num_cores + lax.axis_index("c")
        base = wid * b_per_w
        pltpu.sync_copy(idx_hbm.at[pl.ds(base, b_per_w)], idx_v)
        pltpu.async_copy(table_hbm.at[idx_v], rows_v, sem).wait()
        pltpu.sync_copy(rows_v, out_hbm.at[pl.ds(base, b_per_w)])

    return gather_kernel(table, idx)


def _tc_add(X, rows, bb=32, nb=4):
    B, H, T = X.shape
    TA = 128
    TB = T - TA
    nc = B // bb

    def body(x_hbm, r_vmem, o_hbm, xba, xbb, oba, obb, isem, osem):
        def in_copies(c, s):
            return [
                pltpu.make_async_copy(
                    x_hbm.at[pl.ds(c * bb, bb), :, pl.ds(0, TA)],
                    xba.at[s], isem.at[s]),
                pltpu.make_async_copy(
                    x_hbm.at[pl.ds(c * bb, bb), :, pl.ds(TA, TB)],
                    xbb.at[s], isem.at[s]),
            ]

        def out_copies(c, s):
            return [
                pltpu.make_async_copy(
                    oba.at[s],
                    o_hbm.at[pl.ds(c * bb, bb), :, pl.ds(0, TA)],
                    osem.at[s]),
                pltpu.make_async_copy(
                    obb.at[s],
                    o_hbm.at[pl.ds(c * bb, bb), :, pl.ds(TA, TB)],
                    osem.at[s]),
            ]

        def start_in(c, s):
            for cp in in_copies(c, s):
                cp.start()

        for c in range(min(nb, nc)):
            start_in(c, c)

        def step(c, _):
            s = lax.rem(c, nb)
            for cp in in_copies(c, s):
                cp.wait()

            @pl.when(c >= nb)
            def _():
                for cp in out_copies(c - nb, s):
                    cp.wait()

            r = r_vmem[pl.ds(pl.multiple_of(c * bb, bb), bb), :]
            oba[s] = xba[s] + r[:, :, None]
            obb[s] = xbb[s] + r[:, :, None]

            @pl.when(c + nb < nc)
            def _():
                start_in(c + nb, s)

            for cp in out_copies(c, s):
                cp.start()
            return 0

        lax.fori_loop(0, nc, step, 0)

        for c in range(max(0, nc - nb), nc):
            for cp in out_copies(c, c % nb):
                cp.wait()

    return pl.pallas_call(
        body,
        in_specs=[
            pl.BlockSpec(memory_space=pl.ANY),
            pl.BlockSpec(memory_space=pltpu.VMEM),
        ],
        out_specs=pl.BlockSpec(memory_space=pl.ANY),
        out_shape=jax.ShapeDtypeStruct((B, H, T), jnp.float32),
        scratch_shapes=[
            pltpu.VMEM((nb, bb, H, TA), jnp.float32),
            pltpu.VMEM((nb, bb, H, TB), jnp.float32),
            pltpu.VMEM((nb, bb, H, TA), jnp.float32),
            pltpu.VMEM((nb, bb, H, TB), jnp.float32),
            pltpu.SemaphoreType.DMA((nb,)),
            pltpu.SemaphoreType.DMA((nb,)),
        ],
    )(X, rows)


@jax.jit
def kernel(X, subject_idx, emb):
    rows = _sc_gather(emb, subject_idx.astype(jnp.int32))
    Xp = jnp.pad(X, ((0, 0), (0, 0), (0, 56)))
    outp = _tc_add(Xp, rows)
    return outp[:, :, :200]


# speedup vs baseline: 1.1519x; 1.1519x over previous
import functools

import jax
import jax.numpy as jnp
from jax import lax
from jax.experimental import pallas as pl
from jax.experimental.pallas import tpu as pltpu
from jax.experimental.pallas import tpu_sc as plsc


def _sc_gather(table, idx):
    V, D = table.shape
    (B,) = idx.shape
    info = plsc.get_sparse_core_info()
    nw = info.num_cores * info.num_subcores
    b_per_w = B // nw
    mesh = plsc.VectorSubcoreMesh(core_axis_name="c", subcore_axis_name="s")

    @functools.partial(
        pl.kernel,
        mesh=mesh,
        out_type=jax.ShapeDtypeStruct((B, D), jnp.float32),
        scratch_types=[
            pltpu.VMEM((b_per_w,), jnp.int32),
            pltpu.VMEM((b_per_w, D), jnp.float32),
            pltpu.SemaphoreType.DMA,
        ],
    )
    def gather_kernel(table_hbm, idx_hbm, out_hbm, idx_v, rows_v, sem):
        wid = lax.axis_index("s") * info.num_cores + lax.axis_index("c")
        base = wid * b_per_w
        pltpu.sync_copy(idx_hbm.at[pl.ds(base, b_per_w)], idx_v)
        pltpu.async_copy(table_hbm.at[idx_v], rows_v, sem).wait()
        pltpu.sync_copy(rows_v, out_hbm.at[pl.ds(base, b_per_w)])

    return gather_kernel(table, idx)


def _tc_add(X, rows, bb=32, nb=4):
    B, H, T = X.shape
    TA = 128
    TB = T - TA
    nc = B // bb

    def body(x_hbm, r_vmem, o_hbm, xba, xbb, oba, obb, isem, osem):
        def in_copies(c, s):
            return [
                pltpu.make_async_copy(
                    x_hbm.at[pl.ds(c * bb, bb), :, pl.ds(0, TA)],
                    xba.at[s], isem.at[s]),
                pltpu.make_async_copy(
                    x_hbm.at[pl.ds(c * bb, bb), :, pl.ds(TA, TB)],
                    xbb.at[s], isem.at[s]),
            ]

        def out_copies(c, s):
            return [
                pltpu.make_async_copy(
                    oba.at[s],
                    o_hbm.at[pl.ds(c * bb, bb), :, pl.ds(0, TA)],
                    osem.at[s]),
                pltpu.make_async_copy(
                    obb.at[s],
                    o_hbm.at[pl.ds(c * bb, bb), :, pl.ds(TA, TB)],
                    osem.at[s]),
            ]

        def start_in(c, s):
            for cp in in_copies(c, s):
                cp.start()

        for c in range(min(nb, nc)):
            start_in(c, c)

        def step(c, _):
            s = lax.rem(c, nb)
            for cp in in_copies(c, s):
                cp.wait()

            @pl.when(c >= nb)
            def _():
                for cp in out_copies(c - nb, s):
                    cp.wait()

            r = r_vmem[pl.ds(pl.multiple_of(c * bb, bb), bb), :]
            oba[s] = xba[s] + r[:, :, None]
            obb[s] = xbb[s] + r[:, :, None]

            @pl.when(c + nb < nc)
            def _():
                start_in(c + nb, s)

            for cp in out_copies(c, s):
                cp.start()
            return 0

        lax.fori_loop(0, nc, step, 0)

        for c in range(max(0, nc - nb), nc):
            for cp in out_copies(c, c % nb):
                cp.wait()

    return pl.pallas_call(
        body,
        in_specs=[
            pl.BlockSpec(memory_space=pl.ANY),
            pl.BlockSpec(memory_space=pltpu.VMEM),
        ],
        out_specs=pl.BlockSpec(memory_space=pl.ANY),
        out_shape=jax.ShapeDtypeStruct((B, H, T), jnp.float32),
        scratch_shapes=[
            pltpu.VMEM((nb, bb, H, TA), jnp.float32),
            pltpu.VMEM((nb, bb, H, TB), jnp.float32),
            pltpu.VMEM((nb, bb, H, TA), jnp.float32),
            pltpu.VMEM((nb, bb, H, TB), jnp.float32),
            pltpu.SemaphoreType.DMA((nb,)),
            pltpu.SemaphoreType.DMA((nb,)),
        ],
    )(X, rows)


@jax.jit
def kernel(X, subject_idx, emb):
    rows = _sc_gather(emb, subject_idx.astype(jnp.int32))
    return _tc_add(X, rows)
